# Initial kernel scaffold; baseline (speedup 1.0000x reference)
import jax
import jax.numpy as jnp
from jax.experimental import pallas as pl


def kernel(node_states, edge_index, W_proj, w_edge, gamma, beta):
    raise NotImplementedError("write your pallas kernel here")


# speedup vs baseline: 6.6477x; 6.6477x over previous
import functools

import jax
import jax.numpy as jnp
from jax import lax
from jax.experimental import pallas as pl
from jax.experimental.pallas import tpu as pltpu
from jax.experimental.pallas import tpu_sc as plsc

_NC = 2
_NS = 16
_L = 16
_CH = 128


def _k1_body(n, p, x_ref, w_ref, we_ref, h_ref, ab_ref, m_ref):
    d = x_ref.shape[1]
    x = x_ref[...]
    h = lax.dot_general(x, w_ref[...], (((1,), (1,)), ((), ())),
                        preferred_element_type=jnp.float32)
    h_ref[pl.ds(0, n), :] = h
    h_ref[pl.ds(n, p - n), :] = jnp.zeros((p - n, d), jnp.float32)
    ab = lax.dot_general(h, we_ref[...], (((1,), (1,)), ((), ())),
                         preferred_element_type=jnp.float32)
    ab_ref[pl.ds(0, n), :] = ab
    ab_ref[pl.ds(n, p - n), :] = jnp.full((p - n, 2), -1e30, jnp.float32)
    m = jnp.max(ab[:, 0]) + jnp.max(ab[:, 1])
    m_ref[...] = jnp.full((1, _L), m, jnp.float32)


def _make_k2(p, d, nch):
    rpt = p // _NS
    mesh = plsc.VectorSubcoreMesh(core_axis_name="c", subcore_axis_name="s")

    @functools.partial(
        pl.kernel,
        mesh=mesh,
        compiler_params=pltpu.CompilerParams(needs_layout_passes=False),
        out_type=[
            jax.ShapeDtypeStruct((_NC, p, d), jnp.float32),
            jax.ShapeDtypeStruct((_NC, p), jnp.float32),
        ],
        scratch_types=[
            pltpu.VMEM((2 * p,), jnp.float32),
            pltpu.VMEM((8, _CH), jnp.int32),
            pltpu.VMEM((8, _CH), jnp.int32),
            pltpu.VMEM((_CH,), jnp.float32),
            pltpu.VMEM((_L,), jnp.float32),
            pltpu.VMEM((_CH, d), jnp.float32),
            pltpu.VMEM((rpt,), jnp.float32),
            pltpu.VMEM_SHARED((p, d), jnp.float32),
            pltpu.VMEM_SHARED((p,), jnp.float32),
            pltpu.SemaphoreType.DMA,
        ],
    )
    def k2(h_hbm, ab_hbm, m_hbm, src_hbm, dst_hbm, uagg_out, den_out,
           ab_v, src_v, dst_v, attn_v, m_v, rows_v, zero_v,
           uagg_sh, den_sh, sem):
        cid = lax.axis_index("c")
        sid = lax.axis_index("s")
        w = cid * _NS + sid
        pltpu.sync_copy(ab_hbm, ab_v)
        pltpu.sync_copy(m_hbm.at[0], m_v)
        mval = m_v[...]
        z16 = jnp.zeros((_L,), jnp.float32)

        def zrow(i, _):
            r = rows_v.at[i]
            for j in range(d // _L):
                r[pl.ds(j * _L, _L)] = z16
            return 0

        lax.fori_loop(0, _CH, zrow, 0)

        def zvec(i, _):
            zero_v[pl.ds(i * _L, _L)] = z16
            return 0

        lax.fori_loop(0, rpt // _L, zvec, 0)

        base = sid * rpt
        pltpu.sync_copy(zero_v, den_sh.at[pl.ds(base, rpt)])
        for k in range(rpt // _CH):
            pltpu.sync_copy(rows_v, uagg_sh.at[pl.ds(base + k * _CH, _CH)])
        plsc.subcore_barrier()

        zi = jnp.zeros((_L,), jnp.int32)

        def group(g, _):
            pltpu.sync_copy(src_hbm.at[pl.ds(w * nch + g * 8, 8)], src_v)
            pltpu.sync_copy(dst_hbm.at[pl.ds(w * nch + g * 8, 8)], dst_v)
            lax.fori_loop(0, 8, chunk, 0)
            return 0

        def chunk(c, _):
            srow = src_v.at[c]
            drow = dst_v.at[c]
            for j in range(_CH // _L):
                si = srow[pl.ds(j * _L, _L)]
                di = drow[pl.ds(j * _L, _L)]
                ga = plsc.load_gather(ab_v, [si * 2])
                gb = plsc.load_gather(ab_v, [di * 2 + 1])
                s = ga + gb
                s = jnp.where(s >= 0, s, 0.2 * s)
                attn_v[pl.ds(j * _L, _L)] = jnp.exp(s - mval)
            pltpu.async_copy(h_hbm.at[srow], rows_v, sem).wait()

            def scale(i, _):
                wsp = plsc.load_gather(attn_v, [zi + i])
                r = rows_v.at[i]
                for j in range(d // _L):
                    r[pl.ds(j * _L, _L)] = r[pl.ds(j * _L, _L)] * wsp
                return 0

            lax.fori_loop(0, _CH, scale, 0)
            pltpu.sync_copy(attn_v, den_sh.at[drow], add=True)
            pltpu.sync_copy(rows_v, uagg_sh.at[drow], add=True)
            return 0

        lax.fori_loop(0, nch // 8, group, 0)
        plsc.subcore_barrier()
        pltpu.sync_copy(uagg_sh.at[pl.ds(base, rpt)],
                        uagg_out.at[cid, pl.ds(base, rpt)])
        pltpu.sync_copy(den_sh.at[pl.ds(base, rpt)],
                        den_out.at[cid, pl.ds(base, rpt)])

    return k2


def _k3_body(up_ref, dp_ref, x_ref, g_ref, b_ref, o_ref):
    u = up_ref[0] + up_ref[1]
    den = dp_ref[0] + dp_ref[1] + 1e-6
    y = u / den[:, None] + x_ref[...]
    mean = jnp.mean(y, axis=1, keepdims=True)
    yc = y - mean
    var = jnp.mean(yc * yc, axis=1, keepdims=True)
    o = yc * lax.rsqrt(var + 1e-5) * g_ref[...] + b_ref[...]
    o_ref[...] = o * 0.5 * (1.0 + lax.erf(o * 0.7071067811865476))


def kernel(node_states, edge_index, W_proj, w_edge, gamma, beta):
    n, d = node_states.shape
    e = edge_index.shape[1]
    nw = _NC * _NS
    p = -(-(n + 1) // (_NS * _CH)) * (_NS * _CH)
    nch = -(-(-(-e // (nw * _CH))) // 8) * 8
    e_pad = nw * _CH * nch

    we2 = w_edge.reshape(2, d)
    h_ext, ab_ext, m16 = pl.pallas_call(
        functools.partial(_k1_body, n, p),
        out_shape=[
            jax.ShapeDtypeStruct((p, d), jnp.float32),
            jax.ShapeDtypeStruct((p, 2), jnp.float32),
            jax.ShapeDtypeStruct((1, _L), jnp.float32),
        ],
    )(node_states, W_proj, we2)

    pad = jnp.full((e_pad - e,), n, jnp.int32)
    src_p = jnp.concatenate([edge_index[0], pad]).reshape(nw * nch, _CH)
    dst_p = jnp.concatenate([edge_index[1], pad]).reshape(nw * nch, _CH)

    uagg_p, den_p = _make_k2(p, d, nch)(h_ext, ab_ext.reshape(2 * p), m16,
                                        src_p, dst_p)

    br = 1024
    out = pl.pallas_call(
        _k3_body,
        grid=(pl.cdiv(n, br),),
        in_specs=[
            pl.BlockSpec((_NC, br, d), lambda i: (0, i, 0)),
            pl.BlockSpec((_NC, br), lambda i: (0, i)),
            pl.BlockSpec((br, d), lambda i: (i, 0)),
            pl.BlockSpec((1, d), lambda i: (0, 0)),
            pl.BlockSpec((1, d), lambda i: (0, 0)),
        ],
        out_specs=pl.BlockSpec((br, d), lambda i: (i, 0)),
        out_shape=jax.ShapeDtypeStruct((n, d), jnp.float32),
    )(uagg_p, den_p, node_states, gamma.reshape(1, d), beta.reshape(1, d))
    return out


# speedup vs baseline: 6.9700x; 1.0485x over previous
import functools

import jax
import jax.numpy as jnp
from jax import lax
from jax.experimental import pallas as pl
from jax.experimental.pallas import tpu as pltpu
from jax.experimental.pallas import tpu_sc as plsc

_NC = 2
_NS = 16
_L = 16
_CH = 128


def _k1_body(n, p, x_ref, w_ref, we_ref, h_ref, ab_ref, m_ref):
    d = x_ref.shape[1]
    x = x_ref[...]
    h = lax.dot_general(x, w_ref[...], (((1,), (1,)), ((), ())),
                        preferred_element_type=jnp.float32)
    h_ref[pl.ds(0, n), :] = h
    h_ref[pl.ds(n, p - n), :] = jnp.zeros((p - n, d), jnp.float32)
    ab = lax.dot_general(h, we_ref[...], (((1,), (1,)), ((), ())),
                         preferred_element_type=jnp.float32)
    ab_ref[pl.ds(0, n), :] = ab
    ab_ref[pl.ds(n, p - n), :] = jnp.full((p - n, 2), -1e30, jnp.float32)
    m = jnp.max(ab[:, 0]) + jnp.max(ab[:, 1])
    m_ref[...] = jnp.full((1, _L), m, jnp.float32)


def _make_k2(p, d, nch):
    rpt = p // _NS
    mesh = plsc.VectorSubcoreMesh(core_axis_name="c", subcore_axis_name="s")

    @functools.partial(
        pl.kernel,
        mesh=mesh,
        compiler_params=pltpu.CompilerParams(needs_layout_passes=False),
        out_type=[
            jax.ShapeDtypeStruct((_NC, p, d), jnp.float32),
            jax.ShapeDtypeStruct((_NC, p), jnp.float32),
        ],
        scratch_types=[
            pltpu.VMEM((2 * p,), jnp.float32),
            pltpu.VMEM((8, _CH), jnp.int32),
            pltpu.VMEM((8, _CH), jnp.int32),
            pltpu.VMEM((_CH,), jnp.float32),
            pltpu.VMEM((_L,), jnp.float32),
            pltpu.VMEM((_CH, d), jnp.float32),
            pltpu.VMEM((rpt,), jnp.float32),
            pltpu.VMEM_SHARED((p, d), jnp.float32),
            pltpu.VMEM_SHARED((p,), jnp.float32),
            pltpu.SemaphoreType.DMA,
        ],
    )
    def k2(h_hbm, ab_hbm, m_hbm, src_hbm, dst_hbm, uagg_out, den_out,
           ab_v, src_v, dst_v, attn_v, m_v, rows_v, zero_v,
           uagg_sh, den_sh, sem):
        cid = lax.axis_index("c")
        sid = lax.axis_index("s")
        w = cid * _NS + sid
        pltpu.sync_copy(ab_hbm, ab_v)
        pltpu.sync_copy(m_hbm.at[0], m_v)
        mval = m_v[...]
        z16 = jnp.zeros((_L,), jnp.float32)

        def zrow(i, _):
            r = rows_v.at[i]
            for j in range(d // _L):
                r[pl.ds(j * _L, _L)] = z16
            return 0

        lax.fori_loop(0, _CH, zrow, 0)

        def zvec(i, _):
            zero_v[pl.ds(i * _L, _L)] = z16
            return 0

        lax.fori_loop(0, rpt // _L, zvec, 0)

        base = sid * rpt
        pltpu.sync_copy(zero_v, den_sh.at[pl.ds(base, rpt)])
        for k in range(rpt // _CH):
            pltpu.sync_copy(rows_v, uagg_sh.at[pl.ds(base + k * _CH, _CH)])
        plsc.subcore_barrier()

        zi = jnp.zeros((_L,), jnp.int32)

        def group(g, _):
            pltpu.sync_copy(src_hbm.at[pl.ds(w * nch + g * 8, 8)], src_v)
            pltpu.sync_copy(dst_hbm.at[pl.ds(w * nch + g * 8, 8)], dst_v)
            lax.fori_loop(0, 8, chunk, 0)
            return 0

        def chunk(c, _):
            srow = src_v.at[c]
            drow = dst_v.at[c]
            cp = pltpu.async_copy(h_hbm.at[srow], rows_v, sem)
            for j in range(_CH // _L):
                si = srow[pl.ds(j * _L, _L)]
                di = drow[pl.ds(j * _L, _L)]
                ga = plsc.load_gather(ab_v, [si * 2])
                gb = plsc.load_gather(ab_v, [di * 2 + 1])
                s = ga + gb
                s = jnp.where(s >= 0, s, 0.2 * s)
                attn_v[pl.ds(j * _L, _L)] = jnp.exp(s - mval)
            pltpu.sync_copy(attn_v, den_sh.at[drow], add=True)
            cp.wait()

            def scale(i4, _):
                for u in range(4):
                    i = i4 * 4 + u
                    wsp = plsc.load_gather(attn_v, [zi + i])
                    r = rows_v.at[i]
                    for j in range(d // _L):
                        r[pl.ds(j * _L, _L)] = r[pl.ds(j * _L, _L)] * wsp
                return 0

            lax.fori_loop(0, _CH // 4, scale, 0)
            pltpu.sync_copy(rows_v, uagg_sh.at[drow], add=True)
            return 0

        lax.fori_loop(0, nch // 8, group, 0)
        plsc.subcore_barrier()
        pltpu.sync_copy(uagg_sh.at[pl.ds(base, rpt)],
                        uagg_out.at[cid, pl.ds(base, rpt)])
        pltpu.sync_copy(den_sh.at[pl.ds(base, rpt)],
                        den_out.at[cid, pl.ds(base, rpt)])

    return k2


def _k3_body(up_ref, dp_ref, x_ref, g_ref, b_ref, o_ref):
    u = up_ref[0] + up_ref[1]
    den = dp_ref[0] + dp_ref[1] + 1e-6
    y = u / den[:, None] + x_ref[...]
    mean = jnp.mean(y, axis=1, keepdims=True)
    yc = y - mean
    var = jnp.mean(yc * yc, axis=1, keepdims=True)
    o = yc * lax.rsqrt(var + 1e-5) * g_ref[...] + b_ref[...]
    o_ref[...] = o * 0.5 * (1.0 + lax.erf(o * 0.7071067811865476))


def kernel(node_states, edge_index, W_proj, w_edge, gamma, beta):
    n, d = node_states.shape
    e = edge_index.shape[1]
    nw = _NC * _NS
    p = -(-(n + 1) // (_NS * _CH)) * (_NS * _CH)
    nch = -(-(-(-e // (nw * _CH))) // 8) * 8
    e_pad = nw * _CH * nch

    we2 = w_edge.reshape(2, d)
    h_ext, ab_ext, m16 = pl.pallas_call(
        functools.partial(_k1_body, n, p),
        out_shape=[
            jax.ShapeDtypeStruct((p, d), jnp.float32),
            jax.ShapeDtypeStruct((p, 2), jnp.float32),
            jax.ShapeDtypeStruct((1, _L), jnp.float32),
        ],
    )(node_states, W_proj, we2)

    pad = jnp.full((e_pad - e,), n, jnp.int32)
    src_p = jnp.concatenate([edge_index[0], pad]).reshape(nw * nch, _CH)
    dst_p = jnp.concatenate([edge_index[1], pad]).reshape(nw * nch, _CH)

    uagg_p, den_p = _make_k2(p, d, nch)(h_ext, ab_ext.reshape(2 * p), m16,
                                        src_p, dst_p)

    br = 1024
    out = pl.pallas_call(
        _k3_body,
        grid=(pl.cdiv(n, br),),
        in_specs=[
            pl.BlockSpec((_NC, br, d), lambda i: (0, i, 0)),
            pl.BlockSpec((_NC, br), lambda i: (0, i)),
            pl.BlockSpec((br, d), lambda i: (i, 0)),
            pl.BlockSpec((1, d), lambda i: (0, 0)),
            pl.BlockSpec((1, d), lambda i: (0, 0)),
        ],
        out_specs=pl.BlockSpec((br, d), lambda i: (i, 0)),
        out_shape=jax.ShapeDtypeStruct((n, d), jnp.float32),
    )(uagg_p, den_p, node_states, gamma.reshape(1, d), beta.reshape(1, d))
    return out


# speedup vs baseline: 13.1991x; 1.8937x over previous
import functools

import jax
import jax.numpy as jnp
from jax import lax
from jax.experimental import pallas as pl
from jax.experimental.pallas import tpu as pltpu
from jax.experimental.pallas import tpu_sc as plsc

_NC = 2
_NS = 16
_L = 16
_CH = 128


def _k1_body(n, p, x_ref, w_ref, we_ref, h16_ref, ab_ref, m_ref):
    d = x_ref.shape[1]
    x = x_ref[...]
    h = lax.dot_general(x, w_ref[...], (((1,), (1,)), ((), ())),
                        preferred_element_type=jnp.float32)
    h16_ref[pl.ds(0, n), :] = h.astype(jnp.bfloat16)
    h16_ref[pl.ds(n, p - n), :] = jnp.zeros((p - n, d), jnp.bfloat16)
    ab = lax.dot_general(h, we_ref[...], (((1,), (1,)), ((), ())),
                         preferred_element_type=jnp.float32)
    ab_ref[pl.ds(0, n), :] = ab
    ab_ref[pl.ds(n, p - n), :] = jnp.full((p - n, 2), -1e30, jnp.float32)
    m = jnp.max(ab[:, 0]) + jnp.max(ab[:, 1])
    m_ref[...] = jnp.full((1, _L), m, jnp.float32)


def _make_k2(p, d, nch):
    rpt = p // _NS
    mesh = plsc.VectorSubcoreMesh(core_axis_name="c", subcore_axis_name="s")

    @functools.partial(
        pl.kernel,
        mesh=mesh,
        compiler_params=pltpu.CompilerParams(needs_layout_passes=False,
                                             use_tc_tiling_on_sc=False),
        out_type=[
            jax.ShapeDtypeStruct((_NC, p, d), jnp.bfloat16),
            jax.ShapeDtypeStruct((_NC, p), jnp.float32),
        ],
        scratch_types=[
            pltpu.VMEM((2 * p,), jnp.float32),
            pltpu.VMEM((nch, _CH), jnp.int32),
            pltpu.VMEM((nch, _CH), jnp.int32),
            pltpu.VMEM((_CH,), jnp.float32),
            pltpu.VMEM((_L,), jnp.float32),
            pltpu.VMEM((_CH, d), jnp.bfloat16),
            pltpu.VMEM((_CH, d), jnp.bfloat16),
            pltpu.VMEM((rpt,), jnp.float32),
            pltpu.VMEM_SHARED((p, d), jnp.bfloat16),
            pltpu.VMEM_SHARED((p,), jnp.float32),
            pltpu.SemaphoreType.DMA,
            pltpu.SemaphoreType.DMA,
        ],
    )
    def k2(h_hbm, ab_hbm, m_hbm, src_hbm, dst_hbm, uagg_out, den_out,
           ab_v, src_v, dst_v, attn_v, m_v, rows_a, rows_b, zero_v,
           uagg_sh, den_sh, sem_a, sem_b):
        cid = lax.axis_index("c")
        sid = lax.axis_index("s")
        w = cid * _NS + sid
        pltpu.sync_copy(ab_hbm, ab_v)
        pltpu.sync_copy(m_hbm.at[0], m_v)
        pltpu.sync_copy(src_hbm.at[pl.ds(w * nch, nch)], src_v)
        pltpu.sync_copy(dst_hbm.at[pl.ds(w * nch, nch)], dst_v)
        mval = m_v[...]
        z32 = jnp.zeros((2 * _L,), jnp.bfloat16)

        def zrow(i, _):
            r = rows_a.at[i]
            for j in range(d // (2 * _L)):
                r[pl.ds(j * 2 * _L, 2 * _L)] = z32
            return 0

        lax.fori_loop(0, _CH, zrow, 0)
        z16 = jnp.zeros((_L,), jnp.float32)

        def zvec(i, _):
            zero_v[pl.ds(i * _L, _L)] = z16
            return 0

        lax.fori_loop(0, rpt // _L, zvec, 0)

        base = sid * rpt
        pltpu.sync_copy(zero_v, den_sh.at[pl.ds(base, rpt)])
        for k in range(rpt // _CH):
            pltpu.sync_copy(rows_a, uagg_sh.at[pl.ds(base + k * _CH, _CH)])
        plsc.subcore_barrier()

        zi = jnp.zeros((_L,), jnp.int32)

        def do_attn(c):
            srow = src_v.at[c]
            drow = dst_v.at[c]
            for j in range(_CH // _L):
                si = srow[pl.ds(j * _L, _L)]
                di = drow[pl.ds(j * _L, _L)]
                ga = plsc.load_gather(ab_v, [si * 2])
                gb = plsc.load_gather(ab_v, [di * 2 + 1])
                s = ga + gb
                s = jnp.where(s >= 0, s, 0.2 * s)
                attn_v[pl.ds(j * _L, _L)] = jnp.exp(s - mval)
            pltpu.sync_copy(attn_v, den_sh.at[drow], add=True)

        def scale_scatter(rows_v, c):
            def scale(i4, _):
                for u in range(4):
                    i = i4 * 4 + u
                    wsp = plsc.load_gather(attn_v, [zi + i])
                    r = rows_v.at[i]
                    for j in range(d // (2 * _L)):
                        x32 = r[pl.ds(j * 2 * _L, 2 * _L)]
                        lo, hi = plsc.unpack(
                            x32, format=plsc.PackFormat.INTERLEAVED,
                            preferred_element_type=jnp.float32)
                        r[pl.ds(j * 2 * _L, 2 * _L)] = plsc.pack(
                            lo * wsp, hi * wsp,
                            format=plsc.PackFormat.INTERLEAVED,
                            preferred_element_type=jnp.bfloat16)
                return 0

            lax.fori_loop(0, _CH // 4, scale, 0)
            pltpu.sync_copy(rows_v, uagg_sh.at[dst_v.at[c]], add=True)

        cp0 = pltpu.async_copy(h_hbm.at[src_v.at[0]], rows_a, sem_a)

        def pair(q, _):
            c0 = q * 2
            do_attn(c0)
            cp_b = pltpu.async_copy(h_hbm.at[src_v.at[c0 + 1]], rows_b, sem_b)
            cp_a_wait = pltpu.make_async_copy(h_hbm.at[src_v.at[c0]], rows_a,
                                              sem_a)
            cp_a_wait.wait()
            scale_scatter(rows_a, c0)
            do_attn(c0 + 1)
            cnext = jnp.minimum(c0 + 2, nch - 1)
            pltpu.async_copy(h_hbm.at[src_v.at[cnext]], rows_a, sem_a)
            cp_b.wait()
            scale_scatter(rows_b, c0 + 1)
            return 0

        lax.fori_loop(0, nch // 2, pair, 0)
        pltpu.make_async_copy(h_hbm.at[src_v.at[nch - 1]], rows_a, sem_a).wait()
        plsc.subcore_barrier()
        pltpu.sync_copy(uagg_sh.at[pl.ds(base, rpt)],
                        uagg_out.at[cid, pl.ds(base, rpt)])
        pltpu.sync_copy(den_sh.at[pl.ds(base, rpt)],
                        den_out.at[cid, pl.ds(base, rpt)])

    return k2


def _k3_body(up_ref, dp_ref, x_ref, g_ref, b_ref, o_ref):
    u = up_ref[0].astype(jnp.float32) + up_ref[1].astype(jnp.float32)
    den = dp_ref[0] + dp_ref[1] + 1e-6
    y = u / den[:, None] + x_ref[...]
    mean = jnp.mean(y, axis=1, keepdims=True)
    yc = y - mean
    var = jnp.mean(yc * yc, axis=1, keepdims=True)
    o = yc * lax.rsqrt(var + 1e-5) * g_ref[...] + b_ref[...]
    o_ref[...] = o * 0.5 * (1.0 + lax.erf(o * 0.7071067811865476))


def kernel(node_states, edge_index, W_proj, w_edge, gamma, beta):
    n, d = node_states.shape
    e = edge_index.shape[1]
    nw = _NC * _NS
    p = -(-(n + 1) // (_NS * _CH)) * (_NS * _CH)
    nch = -(-(-(-e // (nw * _CH))) // 8) * 8
    e_pad = nw * _CH * nch

    we2 = w_edge.reshape(2, d)
    h16, ab_ext, m16 = pl.pallas_call(
        functools.partial(_k1_body, n, p),
        out_shape=[
            jax.ShapeDtypeStruct((p, d), jnp.bfloat16),
            jax.ShapeDtypeStruct((p, 2), jnp.float32),
            jax.ShapeDtypeStruct((1, _L), jnp.float32),
        ],
    )(node_states, W_proj, we2)

    pad = jnp.full((e_pad - e,), n, jnp.int32)
    src_p = jnp.concatenate([edge_index[0], pad]).reshape(nw * nch, _CH)
    dst_p = jnp.concatenate([edge_index[1], pad]).reshape(nw * nch, _CH)

    uagg_p, den_p = _make_k2(p, d, nch)(h16, ab_ext.reshape(2 * p), m16,
                                        src_p, dst_p)

    br = 1024
    out = pl.pallas_call(
        _k3_body,
        grid=(pl.cdiv(n, br),),
        in_specs=[
            pl.BlockSpec((_NC, br, d), lambda i: (0, i, 0)),
            pl.BlockSpec((_NC, br), lambda i: (0, i)),
            pl.BlockSpec((br, d), lambda i: (i, 0)),
            pl.BlockSpec((1, d), lambda i: (0, 0)),
            pl.BlockSpec((1, d), lambda i: (0, 0)),
        ],
        out_specs=pl.BlockSpec((br, d), lambda i: (i, 0)),
        out_shape=jax.ShapeDtypeStruct((n, d), jnp.float32),
    )(uagg_p, den_p, node_states, gamma.reshape(1, d), beta.reshape(1, d))
    return out


# speedup vs baseline: 18.9423x; 1.4351x over previous
import functools

import jax
import jax.numpy as jnp
from jax import lax
from jax.experimental import pallas as pl
from jax.experimental.pallas import tpu as pltpu
from jax.experimental.pallas import tpu_sc as plsc

_NC = 2
_NS = 16
_L = 16
_CH = 128


def _k1_body(n, p, x_ref, w_ref, we_ref, h16_ref, ab_ref, m_ref):
    d = x_ref.shape[1]
    x = x_ref[...]
    h = lax.dot_general(x, w_ref[...], (((1,), (1,)), ((), ())),
                        preferred_element_type=jnp.float32)
    h16_ref[pl.ds(0, n), :] = h.astype(jnp.bfloat16)
    h16_ref[pl.ds(n, p - n), :] = jnp.zeros((p - n, d), jnp.bfloat16)
    ab = lax.dot_general(h, we_ref[...], (((1,), (1,)), ((), ())),
                         preferred_element_type=jnp.float32)
    ab_ref[pl.ds(0, n), :] = ab
    ab_ref[pl.ds(n, p - n), :] = jnp.full((p - n, 2), -1e30, jnp.float32)
    m = jnp.max(ab[:, 0]) + jnp.max(ab[:, 1])
    m_ref[...] = jnp.full((1, _L), m, jnp.float32)


def _make_k2(p, d, nch):
    rpt = p // _NS
    mesh = plsc.VectorSubcoreMesh(core_axis_name="c", subcore_axis_name="s")

    @functools.partial(
        pl.kernel,
        mesh=mesh,
        compiler_params=pltpu.CompilerParams(needs_layout_passes=False,
                                             use_tc_tiling_on_sc=False),
        out_type=[
            jax.ShapeDtypeStruct((_NC, p, d), jnp.bfloat16),
            jax.ShapeDtypeStruct((_NC, p), jnp.float32),
        ],
        scratch_types=[
            pltpu.VMEM((2 * p,), jnp.float32),
            pltpu.VMEM((nch, _CH), jnp.int32),
            pltpu.VMEM((nch, _CH), jnp.int32),
            pltpu.VMEM((_CH,), jnp.float32),
            pltpu.VMEM((_L,), jnp.float32),
            pltpu.VMEM((_CH, d), jnp.bfloat16),
            pltpu.VMEM((_CH, d), jnp.bfloat16),
            pltpu.VMEM((rpt,), jnp.float32),
            pltpu.VMEM_SHARED((p, d), jnp.bfloat16),
            pltpu.VMEM_SHARED((p,), jnp.float32),
            pltpu.SemaphoreType.DMA,
            pltpu.SemaphoreType.DMA,
        ],
    )
    def k2(h_hbm, ab_hbm, m_hbm, src_hbm, dst_hbm, uagg_out, den_out,
           ab_v, src_v, dst_v, attn_v, m_v, rows_a, rows_b, zero_v,
           uagg_sh, den_sh, sem_a, sem_b):
        cid = lax.axis_index("c")
        sid = lax.axis_index("s")
        w = cid * _NS + sid
        pltpu.sync_copy(ab_hbm, ab_v)
        pltpu.sync_copy(m_hbm.at[0], m_v)
        pltpu.sync_copy(src_hbm.at[pl.ds(w * nch, nch)], src_v)
        pltpu.sync_copy(dst_hbm.at[pl.ds(w * nch, nch)], dst_v)
        mval = m_v[...]
        z32 = jnp.zeros((2 * _L,), jnp.bfloat16)

        def zrow(i, _):
            r = rows_a.at[i]
            for j in range(d // (2 * _L)):
                r[pl.ds(j * 2 * _L, 2 * _L)] = z32
            return 0

        lax.fori_loop(0, _CH, zrow, 0)
        z16 = jnp.zeros((_L,), jnp.float32)

        def zvec(i, _):
            zero_v[pl.ds(i * _L, _L)] = z16
            return 0

        lax.fori_loop(0, rpt // _L, zvec, 0)

        base = sid * rpt
        pltpu.sync_copy(zero_v, den_sh.at[pl.ds(base, rpt)])
        for k in range(rpt // _CH):
            pltpu.sync_copy(rows_a, uagg_sh.at[pl.ds(base + k * _CH, _CH)])
        plsc.subcore_barrier()

        zi = jnp.zeros((_L,), jnp.int32)

        def do_attn(c):
            srow = src_v.at[c]
            drow = dst_v.at[c]
            for j in range(_CH // _L):
                si = srow[pl.ds(j * _L, _L)]
                di = drow[pl.ds(j * _L, _L)]
                ga = plsc.load_gather(ab_v, [si * 2])
                gb = plsc.load_gather(ab_v, [di * 2 + 1])
                s = ga + gb
                s = jnp.where(s >= 0, s, 0.2 * s)
                attn_v[pl.ds(j * _L, _L)] = jnp.exp(s - mval)
            pltpu.sync_copy(attn_v, den_sh.at[drow], add=True)

        def scale_scatter(rows_v, c):
            def scale(i4, _):
                for u in range(4):
                    i = i4 * 4 + u
                    wsp = plsc.load_gather(attn_v, [zi + i])
                    r = rows_v.at[i]
                    for j in range(d // (2 * _L)):
                        x32 = r[pl.ds(j * 2 * _L, 2 * _L)]
                        lo, hi = plsc.unpack(
                            x32, format=plsc.PackFormat.INTERLEAVED,
                            preferred_element_type=jnp.float32)
                        r[pl.ds(j * 2 * _L, 2 * _L)] = plsc.pack(
                            lo * wsp, hi * wsp,
                            format=plsc.PackFormat.INTERLEAVED,
                            preferred_element_type=jnp.bfloat16)
                return 0

            lax.fori_loop(0, _CH // 4, scale, 0)
            pltpu.sync_copy(rows_v, uagg_sh.at[dst_v.at[c]], add=True)

        cp0 = pltpu.async_copy(h_hbm.at[src_v.at[0]], rows_a, sem_a)

        def pair(q, _):
            c0 = q * 2
            do_attn(c0)
            cp_b = pltpu.async_copy(h_hbm.at[src_v.at[c0 + 1]], rows_b, sem_b)
            cp_a_wait = pltpu.make_async_copy(h_hbm.at[src_v.at[c0]], rows_a,
                                              sem_a)
            cp_a_wait.wait()
            scale_scatter(rows_a, c0)
            do_attn(c0 + 1)
            cnext = jnp.minimum(c0 + 2, nch - 1)
            pltpu.async_copy(h_hbm.at[src_v.at[cnext]], rows_a, sem_a)
            cp_b.wait()
            scale_scatter(rows_b, c0 + 1)
            return 0

        lax.fori_loop(0, nch // 2, pair, 0)
        pltpu.make_async_copy(h_hbm.at[src_v.at[nch - 1]], rows_a, sem_a).wait()
        plsc.subcore_barrier()
        pltpu.sync_copy(uagg_sh.at[pl.ds(base, rpt)],
                        uagg_out.at[cid, pl.ds(base, rpt)])
        pltpu.sync_copy(den_sh.at[pl.ds(base, rpt)],
                        den_out.at[cid, pl.ds(base, rpt)])

    return k2


def _k3_body(up_ref, dp_ref, x_ref, g_ref, b_ref, o_ref):
    u = up_ref[0].astype(jnp.float32) + up_ref[1].astype(jnp.float32)
    den = dp_ref[0] + dp_ref[1] + 1e-6
    y = u / den[:, None] + x_ref[...]
    mean = jnp.mean(y, axis=1, keepdims=True)
    yc = y - mean
    var = jnp.mean(yc * yc, axis=1, keepdims=True)
    o = yc * lax.rsqrt(var + 1e-5) * g_ref[...] + b_ref[...]
    o_ref[...] = o * 0.5 * (1.0 + lax.erf(o * 0.7071067811865476))


def kernel(node_states, edge_index, W_proj, w_edge, gamma, beta):
    n, d = node_states.shape
    e = edge_index.shape[1]
    nw = _NC * _NS
    p = -(-(n + 1) // (_NS * _CH)) * (_NS * _CH)
    nch = -(-(-(-e // (nw * _CH))) // 8) * 8
    e_pad = nw * _CH * nch

    we2 = w_edge.reshape(2, d)
    h16, ab_ext, m16 = pl.pallas_call(
        functools.partial(_k1_body, n, p),
        out_shape=[
            jax.ShapeDtypeStruct((p, d), jnp.bfloat16),
            jax.ShapeDtypeStruct((p, 2), jnp.float32),
            jax.ShapeDtypeStruct((1, _L), jnp.float32),
        ],
    )(node_states, W_proj, we2)

    pad = n + jnp.arange(e_pad - e, dtype=jnp.int32) % (p - n)
    src_p = jnp.concatenate([edge_index[0], pad]).reshape(nw * nch, _CH)
    dst_p = jnp.concatenate([edge_index[1], pad]).reshape(nw * nch, _CH)

    uagg_p, den_p = _make_k2(p, d, nch)(h16, ab_ext.reshape(2 * p), m16,
                                        src_p, dst_p)

    br = 1024
    out = pl.pallas_call(
        _k3_body,
        grid=(pl.cdiv(n, br),),
        in_specs=[
            pl.BlockSpec((_NC, br, d), lambda i: (0, i, 0)),
            pl.BlockSpec((_NC, br), lambda i: (0, i)),
            pl.BlockSpec((br, d), lambda i: (i, 0)),
            pl.BlockSpec((1, d), lambda i: (0, 0)),
            pl.BlockSpec((1, d), lambda i: (0, 0)),
        ],
        out_specs=pl.BlockSpec((br, d), lambda i: (i, 0)),
        out_shape=jax.ShapeDtypeStruct((n, d), jnp.float32),
    )(uagg_p, den_p, node_states, gamma.reshape(1, d), beta.reshape(1, d))
    return out
